# baseline (speedup 1.0000x reference)
import functools

import jax
import jax.numpy as jnp
import numpy as np
from jax import lax
from jax.experimental import pallas as pl
from jax.experimental.pallas import tpu as pltpu
from jax.experimental.pallas import tpu_sc as plsc

_W = 64
_C = 14
_N = 1024
_E = 11
_ROWS = _N * _C
_K = _W * _W
_BLK = 512

_COORD_PAD = np.zeros((2, 16), dtype=np.int32)
_COORD_PAD[:, :_E] = np.array(
    [[12, 12, 8, 7, 12, 9, 10, 2, 1, 3, 4],
     [13, 8, 7, 6, 9, 10, 11, 1, 0, 4, 5]], dtype=np.int32)

_NC, _NS, _L = 2, 16, 16


def _reduce_body(x_ref, val_ref, xc_ref, yc_ref):
    blk = x_ref[...]
    m = jnp.max(blk, axis=1)
    iota = lax.broadcasted_iota(jnp.int32, blk.shape, 1)
    hit = jnp.where(blk == m[:, None], iota, _K)
    idx = jnp.min(hit, axis=1)
    val_ref[...] = m
    xc_ref[...] = (idx % _W).astype(jnp.float32) * (1.0 / _W)
    yc_ref[...] = jnp.round(idx.astype(jnp.float32) * (1.0 / _W)) * (1.0 / _W)


def _maxpool_keypoints(flat):
    out = jax.ShapeDtypeStruct((_ROWS,), jnp.float32)
    return pl.pallas_call(
        _reduce_body,
        grid=(_ROWS // _BLK,),
        in_specs=[pl.BlockSpec((_BLK, _K), lambda i: (i, 0))],
        out_specs=[pl.BlockSpec((_BLK,), lambda i: (i,))] * 3,
        out_shape=[out, out, out],
    )(flat)


def _edge_index_sc(coord):
    mesh = plsc.VectorSubcoreMesh(core_axis_name="c", subcore_axis_name="s")
    rows_per_subcore = _N // _NS

    @functools.partial(
        pl.kernel,
        mesh=mesh,
        out_type=jax.ShapeDtypeStruct((2, _N * _L), jnp.int32),
        scratch_types=[
            pltpu.VMEM((_L,), jnp.int32),
            pltpu.VMEM((rows_per_subcore * _L,), jnp.int32),
            pltpu.SemaphoreType.DMA,
            pltpu.SemaphoreType.DMA,
        ],
    )
    def k(coord_hbm, out_hbm, crow, buf, sem_in, sem_out):
        c = lax.axis_index("c")
        s = lax.axis_index("s")
        pltpu.async_copy(coord_hbm.at[c], crow, sem_in).wait()
        base = s * rows_per_subcore

        @pl.loop(0, rows_per_subcore)
        def _(j):
            buf[pl.ds(j * _L, _L)] = crow[...] + (base + j) * _C

        pltpu.async_copy(
            buf, out_hbm.at[c].at[pl.ds(base * _L, rows_per_subcore * _L)],
            sem_out).wait()

    return k(coord)


def kernel(x):
    n, c, w, h = x.shape
    flat = x.reshape(n * c, w * h)
    value, xc, yc = _maxpool_keypoints(flat)
    feature = jnp.stack([value, xc, yc], axis=-1)
    coord = jnp.asarray(_COORD_PAD[:, :_E])
    offsets = jnp.arange(_N, dtype=jnp.int32) * _C
    edge_index = (coord[:, None, :] + offsets[None, :, None]).reshape(2, _N * _E)
    return feature, edge_index


# speedup vs baseline: 1.0539x; 1.0539x over previous
import functools

import jax
import jax.numpy as jnp
import numpy as np
from jax import lax
from jax.experimental import pallas as pl
from jax.experimental.pallas import tpu as pltpu
from jax.experimental.pallas import tpu_sc as plsc

_W = 64
_C = 14
_N = 1024
_E = 11
_ROWS = _N * _C
_K = _W * _W
_BN = 16

_COORD_PAD = np.zeros((2, 16), dtype=np.int32)
_COORD_PAD[:, :_E] = np.array(
    [[12, 12, 8, 7, 12, 9, 10, 2, 1, 3, 4],
     [13, 8, 7, 6, 9, 10, 11, 1, 0, 4, 5]], dtype=np.int32)

_NC, _NS, _L = 2, 16, 16


def _reduce_body(x_ref, val_ref, xc_ref, yc_ref):
    blk = x_ref[...]
    m = jnp.max(blk, axis=(2, 3))
    r = lax.broadcasted_iota(jnp.int32, blk.shape, 2)
    col = lax.broadcasted_iota(jnp.int32, blk.shape, 3)
    flat_pos = r * _W + col
    hit = jnp.where(blk == m[:, :, None, None], flat_pos, _K)
    idx = jnp.min(hit, axis=(2, 3))
    val_ref[...] = m
    xc_ref[...] = (idx % _W).astype(jnp.float32) * (1.0 / _W)
    yc_ref[...] = jnp.round(idx.astype(jnp.float32) * (1.0 / _W)) * (1.0 / _W)


def _maxpool_keypoints(x):
    out = jax.ShapeDtypeStruct((_N, _C), jnp.float32)
    return pl.pallas_call(
        _reduce_body,
        grid=(_N // _BN,),
        in_specs=[pl.BlockSpec((_BN, _C, _W, _W), lambda i: (i, 0, 0, 0))],
        out_specs=[pl.BlockSpec((_BN, _C), lambda i: (i, 0))] * 3,
        out_shape=[out, out, out],
    )(x)


def _edge_index_sc(coord):
    mesh = plsc.VectorSubcoreMesh(core_axis_name="c", subcore_axis_name="s")
    rows_per_subcore = _N // _NS

    @functools.partial(
        pl.kernel,
        mesh=mesh,
        out_type=jax.ShapeDtypeStruct((2, _N * _L), jnp.int32),
        scratch_types=[
            pltpu.VMEM((_L,), jnp.int32),
            pltpu.VMEM((rows_per_subcore * _L,), jnp.int32),
            pltpu.SemaphoreType.DMA,
            pltpu.SemaphoreType.DMA,
        ],
    )
    def k(coord_hbm, out_hbm, crow, buf, sem_in, sem_out):
        c = lax.axis_index("c")
        s = lax.axis_index("s")
        pltpu.async_copy(coord_hbm.at[c], crow, sem_in).wait()
        base = s * rows_per_subcore

        @pl.loop(0, rows_per_subcore)
        def _(j):
            buf[pl.ds(j * _L, _L)] = crow[...] + (base + j) * _C

        pltpu.async_copy(
            buf, out_hbm.at[c].at[pl.ds(base * _L, rows_per_subcore * _L)],
            sem_out).wait()

    return k(coord)


def kernel(x):
    value, xc, yc = _maxpool_keypoints(x)
    feature = jnp.stack([value, xc, yc], axis=-1).reshape(_ROWS, 3)
    coord = jnp.asarray(_COORD_PAD[:, :_E])
    offsets = jnp.arange(_N, dtype=jnp.int32) * _C
    edge_index = (coord[:, None, :] + offsets[None, :, None]).reshape(2, _N * _E)
    return feature, edge_index
